# Initial kernel scaffold; baseline (speedup 1.0000x reference)
import jax
import jax.numpy as jnp
from jax.experimental import pallas as pl


def kernel(node_features, edge_index, relation_features, last_idx, W1, b1, W2, b2, Wl1, Wr1, att1, bias1, Wl2, Wr2, att2, bias2, ln_g, ln_b):
    raise NotImplementedError("write your pallas kernel here")


# speedup vs baseline: 40.7606x; 40.7606x over previous
import functools

import jax
import jax.numpy as jnp
from jax import lax
from jax.experimental import pallas as pl
from jax.experimental.pallas import tpu as pltpu
from jax.experimental.pallas import tpu_sc as plsc

N = 10000
E = 320000
D = 128
B = 16

NC = 2
NS = 16
L = 16

ET = E + N
CHUNK = 32
EPAD = NC * NS * CHUNK * ((ET + NC * NS * CHUNK - 1) // (NC * NS * CHUNK))
E_TILE = EPAD // (NC * NS)
ITERS = E_TILE // CHUNK
ROWS_TILE = N // NS
ZR = 25
ACCW = 144


def _gat_agg_body(heads, ch, src_hbm, dst2d_hbm, xl_hbm, xr_hbm, att_hbm,
                  us_hbm,
                  srcbuf, dstbuf, attbuf, xlrows, xrrows, accrows, zrows,
                  us_sp, sem1, sem2):
    c = lax.axis_index("c")
    s = lax.axis_index("s")
    zero16 = jnp.zeros((L,), jnp.float32)

    def zbody(r, carry):
        for j in range(ACCW // L):
            zrows[r, pl.ds(j * L, L)] = zero16
        return carry
    lax.fori_loop(0, ZR, zbody, 0)
    rbase = s * ROWS_TILE
    for k in range(ROWS_TILE // ZR):
        pltpu.sync_copy(zrows, us_sp.at[pl.ds(rbase + k * ZR, ZR)])

    tile = c * NS + s
    pltpu.sync_copy(src_hbm.at[pl.ds(tile * E_TILE, E_TILE)], srcbuf)
    pltpu.sync_copy(dst2d_hbm.at[pl.ds(tile * ITERS, ITERS)], dstbuf)
    pltpu.sync_copy(att_hbm, attbuf)

    plsc.subcore_barrier()

    attv = [attbuf[pl.ds(j * L, L)] for j in range(heads * ch // L)]
    lanes = lax.iota(jnp.int32, L)
    ebase = tile * E_TILE

    def step(i, carry):
        cp1 = pltpu.async_copy(xl_hbm.at[srcbuf.at[pl.ds(i * CHUNK, CHUNK)]],
                               xlrows, sem1)
        cp2 = pltpu.async_copy(xr_hbm.at[dstbuf.at[i]], xrrows, sem2)
        cp1.wait()
        cp2.wait()
        gbase = ebase + i * CHUNK
        for e in range(CHUNK):
            xlv = [xlrows[e, pl.ds(j * L, L)] for j in range(heads * ch // L)]
            xrv = [xrrows[e, pl.ds(j * L, L)] for j in range(heads * ch // L)]
            wf = jnp.where(gbase + e < ET, 1.0, 0.0)
            wfv = jnp.broadcast_to(wf, (L,))
            srow = zero16
            for h in range(heads):
                acc = None
                for j in range(h * ch // L, (h + 1) * ch // L):
                    t = xlv[j] + xrv[j]
                    p = jnp.maximum(t, 0.2 * t) * attv[j]
                    acc = p if acc is None else acc + p
                logit = jnp.sum(acc)
                evec = jnp.exp(jnp.broadcast_to(logit, (L,))) * wfv
                for j in range(h * ch // L, (h + 1) * ch // L):
                    accrows[e, pl.ds(j * L, L)] = evec * xlv[j]
                srow = jnp.where(lanes == h, evec, srow)
            accrows[e, pl.ds(D, L)] = srow
        pltpu.sync_copy(accrows, us_sp.at[dstbuf.at[i]], add=True)
        return carry

    lax.fori_loop(0, ITERS, step, 0)

    plsc.subcore_barrier()
    pltpu.sync_copy(us_sp.at[pl.ds(rbase, ROWS_TILE)],
                    us_hbm.at[c, pl.ds(rbase, ROWS_TILE)])


def _gat_agg(heads, ch, src_p, dst2d, xl, xr, att):
    mesh = plsc.VectorSubcoreMesh(core_axis_name="c", subcore_axis_name="s",
                                  num_cores=NC, num_subcores=NS)
    f = pl.kernel(
        functools.partial(_gat_agg_body, heads, ch),
        out_type=jax.ShapeDtypeStruct((NC, N, ACCW), jnp.float32),
        mesh=mesh,
        compiler_params=pltpu.CompilerParams(use_tc_tiling_on_sc=False,
                                             needs_layout_passes=False),
        scratch_types=[
            pltpu.VMEM((E_TILE,), jnp.int32),
            pltpu.VMEM((ITERS, CHUNK), jnp.int32),
            pltpu.VMEM((heads * ch,), jnp.float32),
            pltpu.VMEM((CHUNK, heads * ch), jnp.float32),
            pltpu.VMEM((CHUNK, heads * ch), jnp.float32),
            pltpu.VMEM((CHUNK, ACCW), jnp.float32),
            pltpu.VMEM((ZR, ACCW), jnp.float32),
            pltpu.VMEM_SHARED((N, ACCW), jnp.float32),
            pltpu.SemaphoreType.DMA,
            pltpu.SemaphoreType.DMA,
        ],
    )
    return f(src_p, dst2d, xl, xr, att)


def _prep_body(x_ref, rel_ref, w1_ref, b1_ref, w2_ref, b2_ref, wl_ref, wr_ref,
               xl_ref, xr_ref):
    h = jnp.maximum(
        jnp.dot(rel_ref[...], w1_ref[...], preferred_element_type=jnp.float32)
        + b1_ref[...], 0.0)
    z = jnp.sum(h * w2_ref[...], axis=1, keepdims=True) + b2_ref[...]
    wx = x_ref[...] * jax.nn.sigmoid(z)
    xl_ref[...] = jnp.dot(wx, wl_ref[...], preferred_element_type=jnp.float32)
    xr_ref[...] = jnp.dot(wx, wr_ref[...], preferred_element_type=jnp.float32)


def _prep(x, rel, W1, b1, W2, b2, Wl, Wr):
    blk = 1000
    grid = (N // blk,)
    return pl.pallas_call(
        _prep_body,
        grid=grid,
        in_specs=[
            pl.BlockSpec((blk, D), lambda i: (i, 0)),
            pl.BlockSpec((blk, 3), lambda i: (i, 0)),
            pl.BlockSpec((3, 64), lambda i: (0, 0)),
            pl.BlockSpec((1, 64), lambda i: (0, 0)),
            pl.BlockSpec((1, 64), lambda i: (0, 0)),
            pl.BlockSpec((1, 1), lambda i: (0, 0)),
            pl.BlockSpec((D, D), lambda i: (0, 0)),
            pl.BlockSpec((D, D), lambda i: (0, 0)),
        ],
        out_specs=[
            pl.BlockSpec((blk, D), lambda i: (i, 0)),
            pl.BlockSpec((blk, D), lambda i: (i, 0)),
        ],
        out_shape=[
            jax.ShapeDtypeStruct((N, D), jnp.float32),
            jax.ShapeDtypeStruct((N, D), jnp.float32),
        ],
    )(x, rel, W1, b1.reshape(1, 64), W2.reshape(1, 64), b2.reshape(1, 1),
      Wl, Wr)


def _mid_body(u0_ref, u1_ref, s0_ref, s1_ref, bias_ref, wl_ref, wr_ref,
              xl_ref, xr_ref):
    u = u0_ref[...] + u1_ref[...]
    s4 = s0_ref[...] + s1_ref[...]
    ex = jnp.where(
        lax.broadcasted_iota(jnp.int32, (4, D), 1) // 32
        == lax.broadcasted_iota(jnp.int32, (4, D), 0), 1.0, 0.0)
    sden = jnp.dot(s4, ex, preferred_element_type=jnp.float32) + 1e-16
    h1 = u / sden + bias_ref[...]
    h1 = jnp.where(h1 > 0, h1, jnp.exp(h1) - 1.0)
    xl_ref[...] = jnp.dot(h1, wl_ref[...], preferred_element_type=jnp.float32)
    xr_ref[...] = jnp.dot(h1, wr_ref[...], preferred_element_type=jnp.float32)


def _mid(us1, bias1, Wl2, Wr2):
    blk = 1000
    grid = (N // blk,)
    u0 = us1[0, :, :D]
    u1 = us1[1, :, :D]
    s0 = us1[0, :, D:D + 4]
    s1 = us1[1, :, D:D + 4]
    return pl.pallas_call(
        _mid_body,
        grid=grid,
        in_specs=[
            pl.BlockSpec((blk, D), lambda i: (i, 0)),
            pl.BlockSpec((blk, D), lambda i: (i, 0)),
            pl.BlockSpec((blk, 4), lambda i: (i, 0)),
            pl.BlockSpec((blk, 4), lambda i: (i, 0)),
            pl.BlockSpec((1, D), lambda i: (0, 0)),
            pl.BlockSpec((D, D), lambda i: (0, 0)),
            pl.BlockSpec((D, D), lambda i: (0, 0)),
        ],
        out_specs=[
            pl.BlockSpec((blk, D), lambda i: (i, 0)),
            pl.BlockSpec((blk, D), lambda i: (i, 0)),
        ],
        out_shape=[
            jax.ShapeDtypeStruct((N, D), jnp.float32),
            jax.ShapeDtypeStruct((N, D), jnp.float32),
        ],
    )(u0, u1, s0, s1, bias1.reshape(1, D), Wl2, Wr2)


def _final_body(u0_ref, u1_ref, s0_ref, s1_ref, bias_ref, g_ref, b_ref,
                out_ref):
    u = u0_ref[...] + u1_ref[...]
    sden = (s0_ref[...] + s1_ref[...])[:, :1] + 1e-16
    h2 = u / sden + bias_ref[...]
    mu = jnp.mean(h2, axis=1, keepdims=True)
    var = jnp.mean((h2 - mu) ** 2, axis=1, keepdims=True)
    out_ref[...] = (h2 - mu) / jnp.sqrt(var + 1e-5) * g_ref[...] + b_ref[...]


def _final(us2, bias2, ln_g, ln_b):
    blk = 1000
    grid = (N // blk,)
    u0 = us2[0, :, :D]
    u1 = us2[1, :, :D]
    s0 = us2[0, :, D:D + 4]
    s1 = us2[1, :, D:D + 4]
    return pl.pallas_call(
        _final_body,
        grid=grid,
        in_specs=[
            pl.BlockSpec((blk, D), lambda i: (i, 0)),
            pl.BlockSpec((blk, D), lambda i: (i, 0)),
            pl.BlockSpec((blk, 4), lambda i: (i, 0)),
            pl.BlockSpec((blk, 4), lambda i: (i, 0)),
            pl.BlockSpec((1, D), lambda i: (0, 0)),
            pl.BlockSpec((1, D), lambda i: (0, 0)),
            pl.BlockSpec((1, D), lambda i: (0, 0)),
        ],
        out_specs=pl.BlockSpec((blk, D), lambda i: (i, 0)),
        out_shape=jax.ShapeDtypeStruct((N, D), jnp.float32),
    )(u0, u1, s0, s1, bias2.reshape(1, D), ln_g.reshape(1, D),
      ln_b.reshape(1, D))


def kernel(node_features, edge_index, relation_features, last_idx, W1, b1,
           W2, b2, Wl1, Wr1, att1, bias1, Wl2, Wr2, att2, bias2, ln_g, ln_b):
    loop = jnp.arange(N, dtype=edge_index.dtype)
    src = jnp.concatenate([edge_index[0], loop])
    dst = jnp.concatenate([edge_index[1], loop])
    src_p = jnp.pad(src, (0, EPAD - ET))
    dst_p = jnp.pad(dst, (0, EPAD - ET))
    dst2d = dst_p.reshape(-1, CHUNK)

    xl1, xr1 = _prep(node_features, relation_features, W1, b1, W2, b2,
                     Wl1, Wr1)
    us1 = _gat_agg(4, 32, src_p, dst2d, xl1, xr1, att1.reshape(-1))
    xl2, xr2 = _mid(us1, bias1, Wl2, Wr2)
    us2 = _gat_agg(1, D, src_p, dst2d, xl2, xr2, att2.reshape(-1))
    h2 = _final(us2, bias2, ln_g, ln_b)
    return h2[last_idx]


# speedup vs baseline: 155.6253x; 3.8180x over previous
import functools

import jax
import jax.numpy as jnp
from jax import lax
from jax.experimental import pallas as pl
from jax.experimental.pallas import tpu as pltpu
from jax.experimental.pallas import tpu_sc as plsc

N = 10000
E = 320000
D = 128
B = 16

NC = 2
NS = 16
L = 16
NW = NC * NS

EP_TILE = E // NW
SCAN_IT = EP_TILE // L
NODE_TILE = 320
CAP = 10368
CHUNK = 32
ROWS_TILE = N // NS
ZR = 25
ACCW = 144
NPAD = 10240


def _mesh():
    return plsc.VectorSubcoreMesh(core_axis_name="c", subcore_axis_name="s",
                                  num_cores=NC, num_subcores=NS)


def _sc_params():
    return pltpu.CompilerParams(use_tc_tiling_on_sc=False,
                                needs_layout_passes=False)


def _zero_tbl(tbl, nwords):
    z = jnp.zeros((L,), jnp.int32)

    def zb(k, carry):
        tbl[pl.ds(k * L, L)] = z
        return carry
    lax.fori_loop(0, nwords // L, zb, 0)


def _compact_scan(srcbuf, dstbuf, tbl, csrc, cdst, n_iters, off0):
    def step(i, off):
        sv = srcbuf[pl.ds(i * L, L)]
        dv = dstbuf[pl.ds(i * L, L)]
        fl = plsc.load_gather(tbl, [dv])
        m = fl > 0
        pos = off + jnp.cumsum(jnp.where(m, 1, 0)) - 1
        plsc.store_scatter(csrc, [pos], sv, mask=m)
        plsc.store_scatter(cdst, [pos], dv, mask=m)
        return off + plsc.all_reduce_population_count(m)
    return lax.fori_loop(0, n_iters, step, off0)


def _pad32(csrc, cdst, off):
    zi = jnp.zeros((L,), jnp.int32)
    for b in range(2):
        idx = off + lax.iota(jnp.int32, L) + b * L
        plsc.store_scatter(csrc, [idx], zi)
        plsc.store_scatter(cdst, [idx], zi)


def _filter1_body(src_hbm, dst_hbm, last_hbm, slotv_hbm,
                  l2src_hbm, l2dst_hbm, cnt2_hbm, need1_hbm,
                  srcbuf, dstbuf, tbl, csrc, cdst, lastbuf, slotbuf,
                  onesbuf, zbuf, cbuf):
    c = lax.axis_index("c")
    s = lax.axis_index("s")
    t = c * NS + s

    _zero_tbl(tbl, N)
    pltpu.sync_copy(last_hbm, lastbuf)
    pltpu.sync_copy(slotv_hbm, slotbuf)
    plsc.store_scatter(tbl, [lastbuf[...]], slotbuf[...])
    onesbuf[pl.ds(0, L)] = jnp.ones((L,), jnp.int32)

    _zero_tbl(zbuf, 640)
    base = c * N + s * 640

    @pl.when(s < 15)
    def _():
        pltpu.sync_copy(zbuf, need1_hbm.at[pl.ds(base, 640)])

    @pl.when(s == 15)
    def _():
        pltpu.sync_copy(zbuf.at[pl.ds(0, 400)],
                        need1_hbm.at[pl.ds(base, 400)])

    pltpu.sync_copy(src_hbm.at[pl.ds(t * EP_TILE, EP_TILE)], srcbuf)
    pltpu.sync_copy(dst_hbm.at[pl.ds(t * EP_TILE, EP_TILE)], dstbuf)

    plsc.subcore_barrier()

    off = _compact_scan(srcbuf, dstbuf, tbl, csrc, cdst, SCAN_IT,
                        jnp.zeros((L,), jnp.int32))
    _pad32(csrc, cdst, off)

    cbuf[pl.ds(0, L)] = off
    pltpu.sync_copy(cbuf, cnt2_hbm.at[t])
    pltpu.sync_copy(csrc, l2src_hbm.at[t])
    pltpu.sync_copy(cdst, l2dst_hbm.at[t])

    offsc = off[0]
    nk = (offsc + 32 + L - 1) // L

    def nstep(k, carry):
        idxv = csrc[pl.ds(k * L, L)] + c * N
        pltpu.sync_copy(onesbuf, need1_hbm.at[idxv])
        return carry
    lax.fori_loop(0, nk, nstep, 0)

    @pl.when(jnp.logical_and(c == 0, s == 0))
    def _():
        pltpu.sync_copy(onesbuf, need1_hbm.at[lastbuf[...]])


def _filter2_body(src_hbm, dst_hbm, need1_hbm,
                  l1src_hbm, l1dst_hbm, cnt1_hbm,
                  srcbuf, dstbuf, p0buf, p1buf, csrc, cdst, cbuf):
    c = lax.axis_index("c")
    s = lax.axis_index("s")
    t = c * NS + s

    pltpu.sync_copy(need1_hbm.at[pl.ds(0, N)], p0buf.at[pl.ds(0, N)])
    pltpu.sync_copy(need1_hbm.at[pl.ds(N, N)], p1buf.at[pl.ds(0, N)])
    pltpu.sync_copy(src_hbm.at[pl.ds(t * EP_TILE, EP_TILE)], srcbuf)
    pltpu.sync_copy(dst_hbm.at[pl.ds(t * EP_TILE, EP_TILE)], dstbuf)

    def mstep(k, carry):
        p0buf[pl.ds(k * L, L)] = (p0buf[pl.ds(k * L, L)]
                                  | p1buf[pl.ds(k * L, L)])
        return carry
    lax.fori_loop(0, N // L, mstep, 0)

    off = _compact_scan(srcbuf, dstbuf, p0buf, csrc, cdst, SCAN_IT,
                        jnp.zeros((L,), jnp.int32))

    nbase = t * NODE_TILE
    for k in range(NODE_TILE // L):
        n = nbase + k * L + lax.iota(jnp.int32, L)
        fl = p0buf[pl.ds(nbase + k * L, L)]
        m = jnp.logical_and(fl > 0, n < N)
        pos = off + jnp.cumsum(jnp.where(m, 1, 0)) - 1
        plsc.store_scatter(csrc, [pos], n, mask=m)
        plsc.store_scatter(cdst, [pos], n, mask=m)
        off = off + plsc.all_reduce_population_count(m)

    _pad32(csrc, cdst, off)
    cbuf[pl.ds(0, L)] = off
    pltpu.sync_copy(cbuf, cnt1_hbm.at[t])
    pltpu.sync_copy(csrc, l1src_hbm.at[t])
    pltpu.sync_copy(cdst, l1dst_hbm.at[t])


def _edge_batch(heads, ch, xlrows, xrrows, accrows, attv, b, wfs):
    lanes = lax.iota(jnp.int32, L)
    nv = heads * ch // L
    for e in range(b * L, b * L + L):
        xlv = [xlrows[e, pl.ds(j * L, L)] for j in range(nv)]
        xrv = [xrrows[e, pl.ds(j * L, L)] for j in range(nv)]
        wfv = jnp.broadcast_to(wfs[e - b * L], (L,))
        srow = jnp.zeros((L,), jnp.float32)
        for h in range(heads):
            acc = None
            for j in range(h * ch // L, (h + 1) * ch // L):
                tt = xlv[j] + xrv[j]
                p = jnp.maximum(tt, 0.2 * tt) * attv[j]
                acc = p if acc is None else acc + p
            logit = jnp.sum(acc)
            evec = jnp.exp(jnp.broadcast_to(logit, (L,))) * wfv
            for j in range(h * ch // L, (h + 1) * ch // L):
                accrows[e, pl.ds(j * L, L)] = evec * xlv[j]
            srow = jnp.where(lanes == h, evec, srow)
        accrows[e, pl.ds(D, L)] = srow


def _agg_body(heads, ch, trows, translate,
              *refs):
    if translate:
        (esrc_hbm, edst_hbm, cnt_hbm, xl_hbm, xr_hbm, att_hbm,
         last_hbm, slotv_hbm, us_hbm,
         srcbuf, dstbuf, cbuf, attbuf, xlrows, xrrows, accrows, zrows,
         tbl, lastbuf, slotbuf, us_sp, sem1, sem2) = refs
    else:
        (esrc_hbm, edst_hbm, cnt_hbm, xl_hbm, xr_hbm, att_hbm, us_hbm,
         srcbuf, dstbuf, cbuf, attbuf, xlrows, xrrows, accrows, zrows,
         us_sp, sem1, sem2) = refs
    c = lax.axis_index("c")
    s = lax.axis_index("s")
    t = c * NS + s
    zero16 = jnp.zeros((L,), jnp.float32)

    def zbody(r, carry):
        for j in range(ACCW // L):
            zrows[r, pl.ds(j * L, L)] = zero16
        return carry
    lax.fori_loop(0, ZR, zbody, 0)

    if trows == N:
        rbase = s * ROWS_TILE
        for k in range(ROWS_TILE // ZR):
            pltpu.sync_copy(zrows, us_sp.at[pl.ds(rbase + k * ZR, ZR)])
    else:
        @pl.when(s == 0)
        def _():
            pltpu.sync_copy(zrows.at[pl.ds(0, L)], us_sp)

    pltpu.sync_copy(esrc_hbm.at[t], srcbuf)
    pltpu.sync_copy(edst_hbm.at[t], dstbuf)
    pltpu.sync_copy(cnt_hbm.at[t], cbuf)
    pltpu.sync_copy(att_hbm, attbuf)
    if translate:
        _zero_tbl(tbl, N)
        pltpu.sync_copy(last_hbm, lastbuf)
        pltpu.sync_copy(slotv_hbm, slotbuf)
        plsc.store_scatter(tbl, [lastbuf[...]], slotbuf[...])

    plsc.subcore_barrier()

    attv = [attbuf[pl.ds(j * L, L)] for j in range(heads * ch // L)]
    cnt = cbuf[pl.ds(0, L)][0]
    iters = (cnt + CHUNK - 1) // CHUNK

    def step(i, carry):
        svs = [srcbuf[pl.ds(i * CHUNK + b * L, L)] for b in range(2)]
        dvs = [dstbuf[pl.ds(i * CHUNK + b * L, L)] for b in range(2)]
        cps = []
        for b in range(2):
            cps.append(pltpu.async_copy(
                xl_hbm.at[svs[b]], xlrows.at[pl.ds(b * L, L)], sem1))
            cps.append(pltpu.async_copy(
                xr_hbm.at[dvs[b]], xrrows.at[pl.ds(b * L, L)], sem2))
        for cp in cps:
            cp.wait()
        for b in range(2):
            wfs = [jnp.where(i * CHUNK + b * L + e < cnt, 1.0, 0.0)
                   for e in range(L)]
            _edge_batch(heads, ch, xlrows, xrrows, accrows, attv, b, wfs)
            if translate:
                ridx = jnp.maximum(plsc.load_gather(tbl, [dvs[b]]) - 1, 0)
            else:
                ridx = dvs[b]
            pltpu.sync_copy(accrows.at[pl.ds(b * L, L)], us_sp.at[ridx],
                            add=True)
        return carry

    lax.fori_loop(0, iters, step, 0)

    if translate:
        @pl.when(jnp.logical_and(c == 0, s == 0))
        def _():
            lastv = lastbuf[...]
            cp1 = pltpu.async_copy(xl_hbm.at[lastv],
                                   xlrows.at[pl.ds(0, L)], sem1)
            cp2 = pltpu.async_copy(xr_hbm.at[lastv],
                                   xrrows.at[pl.ds(0, L)], sem2)
            cp1.wait()
            cp2.wait()
            _edge_batch(heads, ch, xlrows, xrrows, accrows, attv, 0,
                        [jnp.float32(1.0)] * L)
            pltpu.sync_copy(accrows.at[pl.ds(0, L)],
                            us_sp.at[lax.iota(jnp.int32, L)], add=True)

    plsc.subcore_barrier()
    if trows == N:
        rbase = s * ROWS_TILE
        pltpu.sync_copy(us_sp.at[pl.ds(rbase, ROWS_TILE)],
                        us_hbm.at[c, pl.ds(rbase, ROWS_TILE)])
    else:
        @pl.when(s == 0)
        def _():
            pltpu.sync_copy(us_sp, us_hbm.at[c])


def _filter1(src, dst, last_idx, slotvals):
    f = pl.kernel(
        _filter1_body,
        out_type=(
            jax.ShapeDtypeStruct((NW, CAP), jnp.int32),
            jax.ShapeDtypeStruct((NW, CAP), jnp.int32),
            jax.ShapeDtypeStruct((NW, L), jnp.int32),
            jax.ShapeDtypeStruct((NC * N,), jnp.int32),
        ),
        mesh=_mesh(),
        compiler_params=_sc_params(),
        scratch_types=[
            pltpu.VMEM((EP_TILE,), jnp.int32),
            pltpu.VMEM((EP_TILE,), jnp.int32),
            pltpu.VMEM((N,), jnp.int32),
            pltpu.VMEM((CAP,), jnp.int32),
            pltpu.VMEM((CAP,), jnp.int32),
            pltpu.VMEM((L,), jnp.int32),
            pltpu.VMEM((L,), jnp.int32),
            pltpu.VMEM((L,), jnp.int32),
            pltpu.VMEM((640,), jnp.int32),
            pltpu.VMEM((L,), jnp.int32),
        ],
    )
    return f(src, dst, last_idx, slotvals)


def _filter2(src, dst, need1):
    f = pl.kernel(
        _filter2_body,
        out_type=(
            jax.ShapeDtypeStruct((NW, CAP), jnp.int32),
            jax.ShapeDtypeStruct((NW, CAP), jnp.int32),
            jax.ShapeDtypeStruct((NW, L), jnp.int32),
        ),
        mesh=_mesh(),
        compiler_params=_sc_params(),
        scratch_types=[
            pltpu.VMEM((EP_TILE,), jnp.int32),
            pltpu.VMEM((EP_TILE,), jnp.int32),
            pltpu.VMEM((NPAD,), jnp.int32),
            pltpu.VMEM((NPAD,), jnp.int32),
            pltpu.VMEM((CAP,), jnp.int32),
            pltpu.VMEM((CAP,), jnp.int32),
            pltpu.VMEM((L,), jnp.int32),
        ],
    )
    return f(src, dst, need1)


def _agg(heads, ch, trows, translate, esrc, edst, cnt, xl, xr, att,
         last_idx=None, slotvals=None):
    scratch = [
        pltpu.VMEM((CAP,), jnp.int32),
        pltpu.VMEM((CAP,), jnp.int32),
        pltpu.VMEM((L,), jnp.int32),
        pltpu.VMEM((heads * ch,), jnp.float32),
        pltpu.VMEM((CHUNK, heads * ch), jnp.float32),
        pltpu.VMEM((CHUNK, heads * ch), jnp.float32),
        pltpu.VMEM((CHUNK, ACCW), jnp.float32),
        pltpu.VMEM((ZR, ACCW), jnp.float32),
    ]
    args = [esrc, edst, cnt, xl, xr, att]
    if translate:
        scratch += [
            pltpu.VMEM((N,), jnp.int32),
            pltpu.VMEM((L,), jnp.int32),
            pltpu.VMEM((L,), jnp.int32),
        ]
        args += [last_idx, slotvals]
    scratch += [
        pltpu.VMEM_SHARED((trows, ACCW), jnp.float32),
        pltpu.SemaphoreType.DMA,
        pltpu.SemaphoreType.DMA,
    ]
    f = pl.kernel(
        functools.partial(_agg_body, heads, ch, trows, translate),
        out_type=jax.ShapeDtypeStruct((NC, trows, ACCW), jnp.float32),
        mesh=_mesh(),
        compiler_params=_sc_params(),
        scratch_types=scratch,
    )
    return f(*args)


def _prep_body(x_ref, rel_ref, w1_ref, b1_ref, w2_ref, b2_ref, wl_ref, wr_ref,
               xl_ref, xr_ref):
    h = jnp.maximum(
        jnp.dot(rel_ref[...], w1_ref[...], preferred_element_type=jnp.float32)
        + b1_ref[...], 0.0)
    z = jnp.sum(h * w2_ref[...], axis=1, keepdims=True) + b2_ref[...]
    wx = x_ref[...] * jax.nn.sigmoid(z)
    xl_ref[...] = jnp.dot(wx, wl_ref[...], preferred_element_type=jnp.float32)
    xr_ref[...] = jnp.dot(wx, wr_ref[...], preferred_element_type=jnp.float32)


def _prep(x, rel, W1, b1, W2, b2, Wl, Wr):
    blk = 1000
    grid = (N // blk,)
    return pl.pallas_call(
        _prep_body,
        grid=grid,
        in_specs=[
            pl.BlockSpec((blk, D), lambda i: (i, 0)),
            pl.BlockSpec((blk, 3), lambda i: (i, 0)),
            pl.BlockSpec((3, 64), lambda i: (0, 0)),
            pl.BlockSpec((1, 64), lambda i: (0, 0)),
            pl.BlockSpec((1, 64), lambda i: (0, 0)),
            pl.BlockSpec((1, 1), lambda i: (0, 0)),
            pl.BlockSpec((D, D), lambda i: (0, 0)),
            pl.BlockSpec((D, D), lambda i: (0, 0)),
        ],
        out_specs=[
            pl.BlockSpec((blk, D), lambda i: (i, 0)),
            pl.BlockSpec((blk, D), lambda i: (i, 0)),
        ],
        out_shape=[
            jax.ShapeDtypeStruct((N, D), jnp.float32),
            jax.ShapeDtypeStruct((N, D), jnp.float32),
        ],
    )(x, rel, W1, b1.reshape(1, 64), W2.reshape(1, 64), b2.reshape(1, 1),
      Wl, Wr)


def _mid_body(u0_ref, u1_ref, s0_ref, s1_ref, bias_ref, wl_ref, wr_ref,
              xl_ref, xr_ref):
    u = u0_ref[...] + u1_ref[...]
    s4 = s0_ref[...] + s1_ref[...]
    ex = jnp.where(
        lax.broadcasted_iota(jnp.int32, (4, D), 1) // 32
        == lax.broadcasted_iota(jnp.int32, (4, D), 0), 1.0, 0.0)
    sden = jnp.dot(s4, ex, preferred_element_type=jnp.float32) + 1e-16
    h1 = u / sden + bias_ref[...]
    h1 = jnp.where(h1 > 0, h1, jnp.exp(h1) - 1.0)
    xl_ref[...] = jnp.dot(h1, wl_ref[...], preferred_element_type=jnp.float32)
    xr_ref[...] = jnp.dot(h1, wr_ref[...], preferred_element_type=jnp.float32)


def _mid(us1, bias1, Wl2, Wr2):
    blk = 1000
    grid = (N // blk,)
    u0 = us1[0, :, :D]
    u1 = us1[1, :, :D]
    s0 = us1[0, :, D:D + 4]
    s1 = us1[1, :, D:D + 4]
    return pl.pallas_call(
        _mid_body,
        grid=grid,
        in_specs=[
            pl.BlockSpec((blk, D), lambda i: (i, 0)),
            pl.BlockSpec((blk, D), lambda i: (i, 0)),
            pl.BlockSpec((blk, 4), lambda i: (i, 0)),
            pl.BlockSpec((blk, 4), lambda i: (i, 0)),
            pl.BlockSpec((1, D), lambda i: (0, 0)),
            pl.BlockSpec((D, D), lambda i: (0, 0)),
            pl.BlockSpec((D, D), lambda i: (0, 0)),
        ],
        out_specs=[
            pl.BlockSpec((blk, D), lambda i: (i, 0)),
            pl.BlockSpec((blk, D), lambda i: (i, 0)),
        ],
        out_shape=[
            jax.ShapeDtypeStruct((N, D), jnp.float32),
            jax.ShapeDtypeStruct((N, D), jnp.float32),
        ],
    )(u0, u1, s0, s1, bias1.reshape(1, D), Wl2, Wr2)


def _final_body(u0_ref, u1_ref, s0_ref, s1_ref, bias_ref, g_ref, b_ref,
                out_ref):
    u = u0_ref[...] + u1_ref[...]
    sden = (s0_ref[...] + s1_ref[...])[:, :1] + 1e-16
    h2 = u / sden + bias_ref[...]
    mu = jnp.mean(h2, axis=1, keepdims=True)
    var = jnp.mean((h2 - mu) ** 2, axis=1, keepdims=True)
    out_ref[...] = (h2 - mu) / jnp.sqrt(var + 1e-5) * g_ref[...] + b_ref[...]


def _final(us2, bias2, ln_g, ln_b):
    u0 = us2[0, :, :D]
    u1 = us2[1, :, :D]
    s0 = us2[0, :, D:D + 4]
    s1 = us2[1, :, D:D + 4]
    return pl.pallas_call(
        _final_body,
        grid=(1,),
        in_specs=[
            pl.BlockSpec((B, D), lambda i: (0, 0)),
            pl.BlockSpec((B, D), lambda i: (0, 0)),
            pl.BlockSpec((B, 4), lambda i: (0, 0)),
            pl.BlockSpec((B, 4), lambda i: (0, 0)),
            pl.BlockSpec((1, D), lambda i: (0, 0)),
            pl.BlockSpec((1, D), lambda i: (0, 0)),
            pl.BlockSpec((1, D), lambda i: (0, 0)),
        ],
        out_specs=pl.BlockSpec((B, D), lambda i: (0, 0)),
        out_shape=jax.ShapeDtypeStruct((B, D), jnp.float32),
    )(u0, u1, s0, s1, bias2.reshape(1, D), ln_g.reshape(1, D),
      ln_b.reshape(1, D))


def kernel(node_features, edge_index, relation_features, last_idx, W1, b1,
           W2, b2, Wl1, Wr1, att1, bias1, Wl2, Wr2, att2, bias2, ln_g, ln_b):
    src = edge_index[0]
    dst = edge_index[1]
    firstocc = jnp.searchsorted(last_idx, last_idx).astype(jnp.int32)
    slotvals = firstocc + 1

    xl1, xr1 = _prep(node_features, relation_features, W1, b1, W2, b2,
                     Wl1, Wr1)
    l2src, l2dst, cnt2, need1 = _filter1(src, dst, last_idx, slotvals)
    l1src, l1dst, cnt1 = _filter2(src, dst, need1)
    us1 = _agg(4, 32, N, False, l1src, l1dst, cnt1, xl1, xr1,
               att1.reshape(-1))
    xl2, xr2 = _mid(us1, bias1, Wl2, Wr2)
    us2 = _agg(1, D, B, True, l2src, l2dst, cnt2, xl2, xr2,
               att2.reshape(-1), last_idx, slotvals)
    h2 = _final(us2, bias2, ln_g, ln_b)
    return h2[firstocc]
